# baseline (speedup 1.0000x reference)
import jax
import jax.numpy as jnp
from jax import lax
from jax.experimental import pallas as pl
from jax.experimental.pallas import tpu as pltpu
from jax.experimental.pallas import tpu_sc as plsc
import numpy as np

ROWS = 128
N = 32768
L = 16
C = N // L
SEG = 2
CS = C // SEG
BINS = 256
NC, NS = 2, 16
NW = NC * NS
RPW = ROWS // NW

_MININT = np.int32(-(2 ** 31))
_ALLONES = np.int32(-1)


def _digit(k, p):
    if p == 0:
        s = k
    else:
        s = lax.shift_right_logical(k, jnp.full((L,), 8 * p, jnp.int32))
    return jnp.bitwise_and(s, jnp.full((L,), 255, jnp.int32))


def _to_key(b):
    return jnp.where(b < 0, jnp.bitwise_xor(b, _ALLONES),
                     jnp.bitwise_xor(b, _MININT))


def _from_key(k):
    return jnp.where(k < 0, jnp.bitwise_xor(k, _MININT),
                     jnp.bitwise_xor(k, _ALLONES))


def _phi(r):
    return jnp.bitwise_or(
        lax.shift_left(jnp.bitwise_and(r, jnp.full((L,), C - 1, jnp.int32)),
                       jnp.full((L,), 4, jnp.int32)),
        lax.shift_right_logical(r, jnp.full((L,), 11, jnp.int32)))


def _sc_sort_body(xs_hbm, out_hbm, buf_f, key_a, key_b, *hists):
    wid = lax.axis_index("s") * NC + lax.axis_index("c")
    lanes = lax.iota(jnp.int32, L)
    ones = jnp.full((L,), 1, jnp.int32)
    zeros = jnp.zeros((L,), jnp.int32)

    def do_row(rr, _carry):
        row = wid * RPW + rr
        pltpu.sync_copy(xs_hbm.at[row], buf_f)

        for p in range(4):
            src, dst = [(buf_f, key_a), (key_a, key_b),
                        (key_b, key_a), (key_a, buf_f)][p]

            @plsc.parallel_loop(0, BINS, unroll=4)
            def _zero(j):
                for h in hists:
                    h[pl.ds(j * L, L)] = zeros

            def read_key(i, src=src, p=p):
                v = src[pl.ds(i * L, L)]
                return _to_key(v) if p == 0 else v

            @plsc.parallel_loop(0, CS, unroll=8)
            def _hist(i, p=p, src=src):
                for s, h in enumerate(hists):
                    k = read_key(s * CS + i, src, p)
                    idx = _digit(k, p) * L + lanes
                    plsc.addupdate_scatter(h, [idx], ones)

            def scan_body(j, carry):
                sl = pl.ds(j * L, L)
                vs = [h[sl] for h in hists]
                t = vs[0]
                for v in vs[1:]:
                    t = t + v
                incl = plsc.cumsum(t)
                acc = incl - t + carry
                for s, h in enumerate(hists):
                    h[sl] = acc
                    acc = acc + vs[s]
                return carry + jnp.sum(t)
            lax.fori_loop(0, BINS, scan_body, jnp.int32(0), unroll=4)

            def perm_body(i, _, p=p, src=src, dst=dst):
                for s, h in enumerate(hists):
                    k = read_key(s * CS + i, src, p)
                    idx = _digit(k, p) * L + lanes
                    r = plsc.load_gather(h, [idx])
                    plsc.store_scatter(h, [idx], r + ones)
                    if p < 3:
                        plsc.store_scatter(dst, [_phi(r)], k)
                    else:
                        plsc.store_scatter(dst, [r], _from_key(k))
                return 0
            lax.fori_loop(0, CS, perm_body, 0, unroll=4)

        pltpu.sync_copy(buf_f, out_hbm.at[row])
        return 0

    lax.fori_loop(0, RPW, do_row, 0)


_sc_sort = pl.kernel(
    _sc_sort_body,
    out_type=jax.ShapeDtypeStruct((ROWS, N), jnp.int32),
    mesh=plsc.VectorSubcoreMesh(core_axis_name="c", subcore_axis_name="s"),
    compiler_params=pltpu.CompilerParams(needs_layout_passes=False),
    scratch_types=[
        pltpu.VMEM((N,), jnp.int32),
        pltpu.VMEM((N,), jnp.int32),
        pltpu.VMEM((N,), jnp.int32),
    ] + [pltpu.VMEM((BINS * L,), jnp.int32) for _ in range(SEG)],
)


def kernel(xs):
    xs_i = lax.bitcast_convert_type(xs, jnp.int32)
    return lax.bitcast_convert_type(_sc_sort(xs_i), jnp.float32)


# speedup vs baseline: 2.1726x; 2.1726x over previous
import jax
import jax.numpy as jnp
from jax import lax
from jax.experimental import pallas as pl
from jax.experimental.pallas import tpu as pltpu
from jax.experimental.pallas import tpu_sc as plsc
import numpy as np

ROWS = 128
N = 32768
L = 16
C = N // L
SEG = 1
CS = C // SEG
BINS = 256
G = 8
NC, NS = 2, 16
NW = NC * NS
RPW = ROWS // NW

_MININT = np.int32(-(2 ** 31))
_ALLONES = np.int32(-1)


def _digit(k, p):
    if p == 0:
        s = k
    else:
        s = lax.shift_right_logical(k, jnp.full((L,), 8 * p, jnp.int32))
    return jnp.bitwise_and(s, jnp.full((L,), 255, jnp.int32))


def _to_key(b):
    return jnp.where(b < 0, jnp.bitwise_xor(b, _ALLONES),
                     jnp.bitwise_xor(b, _MININT))


def _from_key(k):
    return jnp.where(k < 0, jnp.bitwise_xor(k, _MININT),
                     jnp.bitwise_xor(k, _ALLONES))


def _phi(r):
    return jnp.bitwise_or(
        lax.shift_left(jnp.bitwise_and(r, jnp.full((L,), C - 1, jnp.int32)),
                       jnp.full((L,), 4, jnp.int32)),
        lax.shift_right_logical(r, jnp.full((L,), 11, jnp.int32)))


def _sc_sort_body(xs_hbm, out_hbm, buf_f, key_a, key_b, *hists):
    wid = lax.axis_index("s") * NC + lax.axis_index("c")
    lanes = lax.iota(jnp.int32, L)
    ones = jnp.full((L,), 1, jnp.int32)
    zeros = jnp.zeros((L,), jnp.int32)

    def do_row(rr, _carry):
        row = wid * RPW + rr
        pltpu.sync_copy(xs_hbm.at[row], buf_f)

        for p in range(4):
            src, dst = [(buf_f, key_a), (key_a, key_b),
                        (key_b, key_a), (key_a, buf_f)][p]

            @plsc.parallel_loop(0, BINS, unroll=4)
            def _zero(j):
                for h in hists:
                    h[pl.ds(j * L, L)] = zeros

            def read_key(i, src=src, p=p):
                v = src[pl.ds(i * L, L)]
                return _to_key(v) if p == 0 else v

            @plsc.parallel_loop(0, CS, unroll=8)
            def _hist(i, p=p, src=src):
                for s, h in enumerate(hists):
                    k = read_key(s * CS + i, src, p)
                    idx = _digit(k, p) * L + lanes
                    plsc.addupdate_scatter(h, [idx], ones)

            def scan_body(j, carry):
                sl = pl.ds(j * L, L)
                vs = [h[sl] for h in hists]
                t = vs[0]
                for v in vs[1:]:
                    t = t + v
                incl = plsc.cumsum(t)
                acc = incl - t + carry
                for s, h in enumerate(hists):
                    h[sl] = acc
                    acc = acc + vs[s]
                return carry + jnp.sum(t)
            lax.fori_loop(0, BINS, scan_body, jnp.int32(0), unroll=4)

            h = hists[0]

            def perm_body(g, _, p=p, src=src, dst=dst):
                base = g * G
                ks = [read_key(base + j, src, p) for j in range(G)]
                idxs = [_digit(k, p) * L + lanes for k in ks]
                rs = [plsc.load_gather(h, [idx]) for idx in idxs]
                for j in range(G):
                    c = rs[j]
                    for jp in range(j):
                        c = c + jnp.where(idxs[jp] == idxs[j], ones, zeros)
                    rs[j] = c
                for idx in idxs:
                    plsc.addupdate_scatter(h, [idx], ones)
                for j in range(G):
                    if p < 3:
                        plsc.store_scatter(dst, [_phi(rs[j])], ks[j])
                    else:
                        plsc.store_scatter(dst, [rs[j]], _from_key(ks[j]))
                return 0
            lax.fori_loop(0, C // G, perm_body, 0)

        pltpu.sync_copy(buf_f, out_hbm.at[row])
        return 0

    lax.fori_loop(0, RPW, do_row, 0)


_sc_sort = pl.kernel(
    _sc_sort_body,
    out_type=jax.ShapeDtypeStruct((ROWS, N), jnp.int32),
    mesh=plsc.VectorSubcoreMesh(core_axis_name="c", subcore_axis_name="s"),
    compiler_params=pltpu.CompilerParams(needs_layout_passes=False),
    scratch_types=[
        pltpu.VMEM((N,), jnp.int32),
        pltpu.VMEM((N,), jnp.int32),
        pltpu.VMEM((N,), jnp.int32),
    ] + [pltpu.VMEM((BINS * L,), jnp.int32) for _ in range(SEG)],
)


def kernel(xs):
    xs_i = lax.bitcast_convert_type(xs, jnp.int32)
    return lax.bitcast_convert_type(_sc_sort(xs_i), jnp.float32)
